# Initial kernel scaffold; baseline (speedup 1.0000x reference)
import jax
import jax.numpy as jnp
from jax.experimental import pallas as pl


def kernel(atom_features, state_features, embedding_table):
    raise NotImplementedError("write your pallas kernel here")


# speedup vs baseline: 1.5048x; 1.5048x over previous
import functools

import jax
import jax.numpy as jnp
from jax import lax
from jax.experimental import pallas as pl
from jax.experimental.pallas import tpu as pltpu
from jax.experimental.pallas import tpu_sc as plsc

_DIM = 128
_N = 100000
_NW = 32
_T = 128
_NT = 25
_CHUNK = _T * _NT

_mesh = plsc.VectorSubcoreMesh(core_axis_name="c", subcore_axis_name="s")


@functools.partial(
    pl.kernel,
    out_type=jax.ShapeDtypeStruct((_N, _DIM), jnp.float32),
    mesh=_mesh,
    scratch_types=[
        pltpu.VMEM((_T,), jnp.int32),
        pltpu.VMEM((_T, _DIM), jnp.float32),
        pltpu.SemaphoreType.DMA,
    ],
)
def _sc_gather(idx_hbm, table_hbm, out_hbm, idx_v, rows_v, gsem):
    wid = lax.axis_index("s") * 2 + lax.axis_index("c")
    base = jnp.minimum(wid * _CHUNK, _N - _CHUNK)

    def body(t, carry):
        off = base + t * _T
        pltpu.sync_copy(idx_hbm.at[pl.ds(off, _T)], idx_v)
        pltpu.async_copy(table_hbm.at[idx_v], rows_v, gsem).wait()
        pltpu.sync_copy(rows_v, out_hbm.at[pl.ds(off, _T)])
        return carry

    lax.fori_loop(0, _NT, body, 0)


def kernel(atom_features, state_features, embedding_table):
    atom_embeds = _sc_gather(atom_features, embedding_table)
    return (atom_embeds, state_features)


# speedup vs baseline: 1.5410x; 1.0241x over previous
import functools

import jax
import jax.numpy as jnp
from jax import lax
from jax.experimental import pallas as pl
from jax.experimental.pallas import tpu as pltpu
from jax.experimental.pallas import tpu_sc as plsc

_DIM = 128
_N = 100000
_NW = 32
_T = 128
_NTILES = 782
_LAST = _N - _T
_NT_FULL = 24
_CUT = _NTILES - _NT_FULL * _NW

_mesh = plsc.VectorSubcoreMesh(core_axis_name="c", subcore_axis_name="s")


@functools.partial(
    pl.kernel,
    out_type=jax.ShapeDtypeStruct((_N, _DIM), jnp.float32),
    mesh=_mesh,
    scratch_types=[
        pltpu.VMEM((4, _T), jnp.int32),
        pltpu.VMEM((4, _T, _DIM), jnp.float32),
        pltpu.SemaphoreType.DMA,
        pltpu.SemaphoreType.DMA,
    ],
)
def _sc_gather(idx_hbm, table_hbm, out_hbm, idx_v, rows_v, gsem, osem):
    wid = lax.axis_index("s") * 2 + lax.axis_index("c")

    def tile_base(t):
        return jnp.minimum((t * _NW + wid) * _T, _LAST)

    def load_idx(t, slot):
        pltpu.sync_copy(idx_hbm.at[pl.ds(tile_base(t), _T)], idx_v.at[slot])

    def start_gather(slot):
        pltpu.async_copy(table_hbm.at[idx_v.at[slot]], rows_v.at[slot], gsem)

    def wait_gather(slot):
        pltpu.make_async_copy(table_hbm.at[idx_v.at[slot]], rows_v.at[slot],
                              gsem).wait()

    def start_store(t, slot):
        pltpu.async_copy(rows_v.at[slot], out_hbm.at[pl.ds(tile_base(t), _T)],
                         osem)

    def wait_store(t, slot):
        pltpu.make_async_copy(rows_v.at[slot],
                              out_hbm.at[pl.ds(tile_base(t), _T)],
                              osem).wait()

    for b in range(4):
        load_idx(b, b)
        start_gather(b)
        if b >= 2:
            wait_gather(b - 2)
            start_store(b - 2, b - 2)

    def outer(p, carry):
        for b in range(4):
            t = p * 4 + b
            wait_store(t - 4, b)
            load_idx(t, b)
            start_gather(b)
            wait_gather((b - 2) % 4)
            start_store(t - 2, (b - 2) % 4)
        return carry

    lax.fori_loop(1, _NT_FULL // 4, outer, 0, unroll=False)

    tail = wid < _CUT

    @pl.when(tail)
    def _():
        wait_store(20, 0)
        load_idx(_NT_FULL, 0)
        start_gather(0)

    wait_gather(2)
    start_store(22, 2)
    wait_gather(3)
    start_store(23, 3)

    @pl.when(tail)
    def _():
        wait_gather(0)
        start_store(_NT_FULL, 0)

    for _ in range(4):
        pltpu.make_async_copy(rows_v.at[0], out_hbm.at[pl.ds(0, _T)],
                              osem).wait()


def kernel(atom_features, state_features, embedding_table):
    atom_embeds = _sc_gather(atom_features, embedding_table)
    return (atom_embeds, state_features)


# speedup vs baseline: 5.5639x; 3.6106x over previous
import functools

import jax
import jax.numpy as jnp
from jax import lax
from jax.experimental import pallas as pl
from jax.experimental.pallas import tpu as pltpu
from jax.experimental.pallas import tpu_sc as plsc

_DIM = 128
_N = 100000
_NW = 32
_T = 128
_NTILES = 782
_LAST = _N - _T
_NT_FULL = 24
_CUT = _NTILES - _NT_FULL * _NW

_mesh = plsc.VectorSubcoreMesh(core_axis_name="c", subcore_axis_name="s")


@functools.partial(
    pl.kernel,
    out_type=jax.ShapeDtypeStruct((_N, _DIM), jnp.float32),
    mesh=_mesh,
    scratch_types=[
        pltpu.VMEM((4, _T), jnp.int32),
        pltpu.VMEM((4, _T, _DIM), jnp.float32),
        pltpu.VMEM_SHARED((119, _DIM), jnp.float32),
        pltpu.SemaphoreType.DMA,
        pltpu.SemaphoreType.DMA,
    ],
)
def _sc_gather(idx_hbm, table_hbm, out_hbm, idx_v, rows_v, table_v, gsem,
               osem):
    sid = lax.axis_index("s")
    wid = sid * 2 + lax.axis_index("c")

    @pl.when(sid == 0)
    def _():
        pltpu.sync_copy(table_hbm, table_v)

    plsc.subcore_barrier()

    def tile_base(t):
        return jnp.minimum((t * _NW + wid) * _T, _LAST)

    def load_idx(t, slot):
        pltpu.sync_copy(idx_hbm.at[pl.ds(tile_base(t), _T)], idx_v.at[slot])

    def start_gather(slot):
        pltpu.async_copy(table_v.at[idx_v.at[slot]], rows_v.at[slot], gsem)

    def wait_gather(slot):
        pltpu.make_async_copy(table_v.at[idx_v.at[slot]], rows_v.at[slot],
                              gsem).wait()

    def start_store(t, slot):
        pltpu.async_copy(rows_v.at[slot], out_hbm.at[pl.ds(tile_base(t), _T)],
                         osem)

    def wait_store(t, slot):
        pltpu.make_async_copy(rows_v.at[slot],
                              out_hbm.at[pl.ds(tile_base(t), _T)],
                              osem).wait()

    for b in range(4):
        load_idx(b, b)
        start_gather(b)
        if b >= 2:
            wait_gather(b - 2)
            start_store(b - 2, b - 2)

    def outer(p, carry):
        for b in range(4):
            t = p * 4 + b
            wait_store(t - 4, b)
            load_idx(t, b)
            start_gather(b)
            wait_gather((b - 2) % 4)
            start_store(t - 2, (b - 2) % 4)
        return carry

    lax.fori_loop(1, _NT_FULL // 4, outer, 0, unroll=False)

    tail = wid < _CUT

    @pl.when(tail)
    def _():
        wait_store(20, 0)
        load_idx(_NT_FULL, 0)
        start_gather(0)

    wait_gather(2)
    start_store(22, 2)
    wait_gather(3)
    start_store(23, 3)

    @pl.when(tail)
    def _():
        wait_gather(0)
        start_store(_NT_FULL, 0)

    for _ in range(4):
        pltpu.make_async_copy(rows_v.at[0], out_hbm.at[pl.ds(0, _T)],
                              osem).wait()


def kernel(atom_features, state_features, embedding_table):
    atom_embeds = _sc_gather(atom_features, embedding_table)
    return (atom_embeds, state_features)


# speedup vs baseline: 5.5939x; 1.0054x over previous
import functools

import jax
import jax.numpy as jnp
from jax import lax
from jax.experimental import pallas as pl
from jax.experimental.pallas import tpu as pltpu
from jax.experimental.pallas import tpu_sc as plsc

_DIM = 128
_N = 100000
_NW = 32
_T = 128
_NT = 25
_CHUNK = _T * _NT
_LASTB = _N - _CHUNK

_mesh = plsc.VectorSubcoreMesh(core_axis_name="c", subcore_axis_name="s")


@functools.partial(
    pl.kernel,
    out_type=jax.ShapeDtypeStruct((_N, _DIM), jnp.float32),
    mesh=_mesh,
    scratch_types=[
        pltpu.VMEM((_CHUNK,), jnp.int32),
        pltpu.VMEM((4, _T, _DIM), jnp.float32),
        pltpu.VMEM_SHARED((119, _DIM), jnp.float32),
        pltpu.SemaphoreType.DMA,
        pltpu.SemaphoreType.DMA,
    ],
)
def _sc_gather(idx_hbm, table_hbm, out_hbm, idx_v, rows_v, table_v, gsem,
               osem):
    sid = lax.axis_index("s")
    wid = sid * 2 + lax.axis_index("c")
    base = jnp.minimum(wid * _CHUNK, _LASTB)

    @pl.when(sid == 0)
    def _():
        pltpu.sync_copy(table_hbm, table_v)

    pltpu.sync_copy(idx_hbm.at[pl.ds(base, _CHUNK)], idx_v)
    plsc.subcore_barrier()

    def start_gather(t, slot):
        pltpu.async_copy(table_v.at[idx_v.at[pl.ds(t * _T, _T)]],
                         rows_v.at[slot], gsem)

    def wait_gather(slot):
        pltpu.make_async_copy(table_v.at[idx_v.at[pl.ds(0, _T)]],
                              rows_v.at[slot], gsem).wait()

    def start_store(t, slot):
        pltpu.async_copy(rows_v.at[slot],
                         out_hbm.at[pl.ds(base + t * _T, _T)], osem)

    def wait_store(t, slot):
        pltpu.make_async_copy(rows_v.at[slot],
                              out_hbm.at[pl.ds(base + t * _T, _T)],
                              osem).wait()

    for b in range(4):
        start_gather(b, b)
        if b >= 2:
            wait_gather(b - 2)
            start_store(b - 2, b - 2)

    def outer(p, carry):
        for b in range(4):
            t = p * 4 + b
            wait_store(t - 4, b)
            start_gather(t, b)
            wait_gather((b - 2) % 4)
            start_store(t - 2, (b - 2) % 4)
        return carry

    lax.fori_loop(1, 6, outer, 0, unroll=False)

    wait_store(20, 0)
    start_gather(24, 0)
    wait_gather(2)
    start_store(22, 2)
    wait_gather(3)
    start_store(23, 3)
    wait_gather(0)
    start_store(24, 0)
    for _ in range(4):
        pltpu.make_async_copy(rows_v.at[0], out_hbm.at[pl.ds(0, _T)],
                              osem).wait()


def kernel(atom_features, state_features, embedding_table):
    atom_embeds = _sc_gather(atom_features, embedding_table)
    return (atom_embeds, state_features)
